# baseline (speedup 1.0000x reference)
import functools

import jax
import jax.numpy as jnp
from jax import lax
from jax.experimental import pallas as pl
from jax.experimental.pallas import tpu as pltpu
from jax.experimental.pallas import tpu_sc as plsc

VOCAB = 30522
H = 768
BATCH = 4
L = 2048
EPS = 1e-12

NCHUNK = 4
LC = L // NCHUNK
TOK = BATCH * LC

NC = 2
NS = 16
NW = NC * NS
TPW = TOK // NW
SUB = TPW // 2
WPB = NW // BATCH

TB = LC


def _sc_gather_body(chunk, ids_hbm, word_hbm, out_hbm,
                    idx0_v, idx1_v, buf0, buf1,
                    sem_i0, sem_i1, sem_g0, sem_g1, sem_o0, sem_o1):
    c = lax.axis_index("c")
    s = lax.axis_index("s")
    wid = s * NC + c
    b = lax.shift_right_logical(wid, 3)
    lw = lax.bitwise_and(wid, WPB - 1)
    l_off = chunk * LC + lw * TPW
    base = wid * TPW

    i0 = pltpu.async_copy(ids_hbm.at[b, pl.ds(l_off, SUB)], idx0_v, sem_i0)
    i1 = pltpu.async_copy(ids_hbm.at[b, pl.ds(l_off + SUB, SUB)], idx1_v,
                          sem_i1)
    i0.wait()
    g0 = pltpu.async_copy(word_hbm.at[idx0_v], buf0, sem_g0)
    i1.wait()
    g1 = pltpu.async_copy(word_hbm.at[idx1_v], buf1, sem_g1)
    g0.wait()
    o0 = pltpu.async_copy(buf0, out_hbm.at[pl.ds(base, SUB)], sem_o0)
    g1.wait()
    o1 = pltpu.async_copy(buf1, out_hbm.at[pl.ds(base + SUB, SUB)], sem_o1)
    o0.wait()
    o1.wait()


def _sc_gather(chunk, input_ids, word_embeddings):
    fn = functools.partial(
        pl.kernel,
        mesh=plsc.VectorSubcoreMesh(core_axis_name="c", subcore_axis_name="s"),
        out_type=jax.ShapeDtypeStruct((TOK, H), jnp.float32),
        scratch_types=[
            pltpu.VMEM((SUB,), jnp.int32),
            pltpu.VMEM((SUB,), jnp.int32),
            pltpu.VMEM((SUB, H), jnp.float32),
            pltpu.VMEM((SUB, H), jnp.float32),
        ] + [pltpu.SemaphoreType.DMA] * 6,
    )(functools.partial(_sc_gather_body, chunk))
    return fn(input_ids, word_embeddings)


def _tc_ln_body(w_ref, p_ref, s_ref, prm_ref, o_ref, acc_ref=None):
    del acc_ref
    w = w_ref[...]
    p = p_ref[...]
    i = pl.program_id(0)
    sg = s_ref[pl.ds(i, 1), :].astype(
        jnp.float32)
    prm = prm_ref[...]
    g = prm[0:1, :]
    bb = prm[1:2, :]
    t0 = prm[2:3, :]
    td = prm[3:4, :] - t0
    tmat = lax.dot_general(sg, td, (((0,), (0,)), ((), ())),
                           precision=lax.Precision.HIGHEST,
                           preferred_element_type=jnp.float32)
    e = w + p + t0 + tmat
    mean = jnp.mean(e, axis=1, keepdims=True)
    cen = e - mean
    var = jnp.mean(cen * cen, axis=1, keepdims=True)
    o_ref[0] = cen * lax.rsqrt(var + EPS) * g + bb


def _tc_ln_body_acc(a_ref, w_ref, p_ref, s_ref, prm_ref, o_ref):
    _tc_ln_body(w_ref, p_ref, s_ref, prm_ref, o_ref, acc_ref=a_ref)


def _tc_ln(chunk, acc, wrows, segf, pos, params):
    grid = (BATCH,)
    data_specs = [
        pl.BlockSpec((TB, H), lambda i: (i, 0)),
        pl.BlockSpec((TB, H), lambda i, c=chunk: (c, 0)),
        pl.BlockSpec((BATCH, TB), lambda i, c=chunk: (0, c)),
        pl.BlockSpec((4, H), lambda i: (0, 0)),
    ]
    out_spec = pl.BlockSpec((1, TB, H), lambda i, c=chunk: (i, c, 0))
    out_shape = jax.ShapeDtypeStruct((BATCH, L, H), jnp.float32)
    if chunk == 0:
        return pl.pallas_call(
            _tc_ln_body, grid=grid, in_specs=data_specs,
            out_specs=out_spec, out_shape=out_shape,
        )(wrows, pos, segf, params)
    return pl.pallas_call(
        _tc_ln_body_acc, grid=grid,
        in_specs=[pl.BlockSpec(memory_space=pl.ANY)] + data_specs,
        out_specs=out_spec, out_shape=out_shape,
        input_output_aliases={0: 0},
    )(acc, wrows, pos, segf, params)


@jax.jit
def _embed(input_ids, segment_ids, word_embeddings, position_embeddings,
           token_type_embeddings, ln_gamma, ln_beta):
    pos = position_embeddings[:L]
    params = jnp.concatenate(
        [ln_gamma.reshape(1, H), ln_beta.reshape(1, H),
         token_type_embeddings], axis=0)
    gathered = [_sc_gather(c, input_ids, word_embeddings)
                for c in range(NCHUNK)]
    out = None
    for c in range(NCHUNK):
        out = _tc_ln(c, out, gathered[c], segment_ids, pos, params)
    return out


def kernel(input_ids, segment_ids, word_embeddings, position_embeddings,
           token_type_embeddings, ln_gamma, ln_beta):
    return _embed(input_ids.astype(jnp.int32), segment_ids.astype(jnp.int32),
                  word_embeddings, position_embeddings, token_type_embeddings,
                  ln_gamma, ln_beta)


# speedup vs baseline: 1.0335x; 1.0335x over previous
import functools

import jax
import jax.numpy as jnp
from jax import lax
from jax.experimental import pallas as pl
from jax.experimental.pallas import tpu as pltpu
from jax.experimental.pallas import tpu_sc as plsc

VOCAB = 30522
H = 768
BATCH = 4
L = 2048
EPS = 1e-12

NCHUNK = 4
LC = L // NCHUNK
TOK = BATCH * LC

NC = 2
NS = 16
NW = NC * NS
TPW = TOK // NW
SUB = TPW // 2
WPB = NW // BATCH

TB = LC


def _sc_gather_body(ids_hbm, word_hbm, out_hbm,
                    idx0_v, idx1_v, buf0, buf1,
                    sem_i0, sem_i1, sem_g0, sem_g1, sem_o0, sem_o1):
    c = lax.axis_index("c")
    s = lax.axis_index("s")
    wid = s * NC + c
    base = wid * TPW

    i0 = pltpu.async_copy(ids_hbm.at[pl.ds(base, SUB)], idx0_v, sem_i0)
    i1 = pltpu.async_copy(ids_hbm.at[pl.ds(base + SUB, SUB)], idx1_v,
                          sem_i1)
    i0.wait()
    g0 = pltpu.async_copy(word_hbm.at[idx0_v], buf0, sem_g0)
    i1.wait()
    g1 = pltpu.async_copy(word_hbm.at[idx1_v], buf1, sem_g1)
    g0.wait()
    o0 = pltpu.async_copy(buf0, out_hbm.at[pl.ds(base, SUB)], sem_o0)
    g1.wait()
    o1 = pltpu.async_copy(buf1, out_hbm.at[pl.ds(base + SUB, SUB)], sem_o1)
    o0.wait()
    o1.wait()


def _sc_gather(ids_chunk, word_embeddings):
    fn = functools.partial(
        pl.kernel,
        mesh=plsc.VectorSubcoreMesh(core_axis_name="c", subcore_axis_name="s"),
        out_type=jax.ShapeDtypeStruct((TOK, H), jnp.float32),
        scratch_types=[
            pltpu.VMEM((SUB,), jnp.int32),
            pltpu.VMEM((SUB,), jnp.int32),
            pltpu.VMEM((SUB, H), jnp.float32),
            pltpu.VMEM((SUB, H), jnp.float32),
        ] + [pltpu.SemaphoreType.DMA] * 6,
    )(_sc_gather_body)
    return fn(ids_chunk, word_embeddings)


def _tc_ln_body(w_ref, p_ref, s_ref, prm_ref, o_ref, acc_ref=None):
    del acc_ref
    w = w_ref[...]
    p = p_ref[...]
    sg = s_ref[0].astype(jnp.float32)
    prm = prm_ref[...]
    g = prm[0:1, :]
    bb = prm[1:2, :]
    t0 = prm[2:3, :]
    td = prm[3:4, :] - t0
    e = w + p + t0 + sg * td
    mean = jnp.mean(e, axis=1, keepdims=True)
    cen = e - mean
    var = jnp.mean(cen * cen, axis=1, keepdims=True)
    o_ref[0] = cen * lax.rsqrt(var + EPS) * g + bb


def _tc_ln_body_acc(a_ref, w_ref, p_ref, s_ref, prm_ref, o_ref):
    _tc_ln_body(w_ref, p_ref, s_ref, prm_ref, o_ref, acc_ref=a_ref)


def _tc_ln(chunk, acc, wrows, segf, pos, params):
    grid = (BATCH,)
    data_specs = [
        pl.BlockSpec((TB, H), lambda i: (i, 0)),
        pl.BlockSpec((TB, H), lambda i, c=chunk: (c, 0)),
        pl.BlockSpec((1, TB, 1), lambda i, c=chunk: (i, c, 0)),
        pl.BlockSpec((4, H), lambda i: (0, 0)),
    ]
    out_spec = pl.BlockSpec((1, TB, H), lambda i, c=chunk: (i, c, 0))
    out_shape = jax.ShapeDtypeStruct((BATCH, L, H), jnp.float32)
    if chunk == 0:
        return pl.pallas_call(
            _tc_ln_body, grid=grid, in_specs=data_specs,
            out_specs=out_spec, out_shape=out_shape,
        )(wrows, pos, segf, params)
    return pl.pallas_call(
        _tc_ln_body_acc, grid=grid,
        in_specs=[pl.BlockSpec(memory_space=pl.ANY)] + data_specs,
        out_specs=out_spec, out_shape=out_shape,
        input_output_aliases={0: 0},
    )(acc, wrows, pos, segf, params)


@jax.jit
def _embed(input_ids, segment_ids, word_embeddings, position_embeddings,
           token_type_embeddings, ln_gamma, ln_beta):
    pos = position_embeddings[:L]
    params = jnp.concatenate(
        [ln_gamma.reshape(1, H), ln_beta.reshape(1, H),
         token_type_embeddings], axis=0)
    seg3 = segment_ids.reshape(BATCH, L, 1)
    ids = input_ids.reshape(BATCH, NCHUNK, LC)
    gathered = [
        _sc_gather(ids[:, c, :].reshape(TOK), word_embeddings)
        for c in range(NCHUNK)
    ]
    out = None
    for c in range(NCHUNK):
        out = _tc_ln(c, out, gathered[c], seg3, pos, params)
    return out


def kernel(input_ids, segment_ids, word_embeddings, position_embeddings,
           token_type_embeddings, ln_gamma, ln_beta):
    return _embed(input_ids.astype(jnp.int32), segment_ids.astype(jnp.int32),
                  word_embeddings, position_embeddings, token_type_embeddings,
                  ln_gamma, ln_beta)


# speedup vs baseline: 1.0480x; 1.0141x over previous
import functools

import jax
import jax.numpy as jnp
from jax import lax
from jax.experimental import pallas as pl
from jax.experimental.pallas import tpu as pltpu
from jax.experimental.pallas import tpu_sc as plsc

VOCAB = 30522
H = 768
BATCH = 4
L = 2048
EPS = 1e-12

NCHUNK = 4
LC = L // NCHUNK
TOK = BATCH * LC

NC = 2
NS = 16
NW = NC * NS
TPW = TOK // NW
SUB = TPW // 2
WPB = NW // BATCH

TB = LC


NWIN = 4
WIN = TPW // NWIN


def _sc_gather_body(ids_hbm, word_hbm, out_hbm, idx_v, *rest):
    bufs = rest[:NWIN]
    sem_i = rest[NWIN]
    gsems = rest[NWIN + 1:2 * NWIN + 1]
    osems = rest[2 * NWIN + 1:]
    c = lax.axis_index("c")
    s = lax.axis_index("s")
    wid = s * NC + c
    base = wid * TPW

    pltpu.async_copy(ids_hbm.at[pl.ds(base, TPW)], idx_v, sem_i).wait()
    gs = [
        pltpu.async_copy(
            word_hbm.at[idx_v.at[pl.ds(k * WIN, WIN)]], bufs[k], gsems[k])
        for k in range(NWIN)
    ]
    os = []
    for k in range(NWIN):
        gs[k].wait()
        os.append(pltpu.async_copy(
            bufs[k], out_hbm.at[pl.ds(base + k * WIN, WIN)], osems[k]))
    for o in os:
        o.wait()


def _sc_gather(ids_chunk, word_embeddings):
    fn = functools.partial(
        pl.kernel,
        mesh=plsc.VectorSubcoreMesh(core_axis_name="c", subcore_axis_name="s"),
        out_type=jax.ShapeDtypeStruct((TOK, H), jnp.float32),
        scratch_types=(
            [pltpu.VMEM((TPW,), jnp.int32)]
            + [pltpu.VMEM((WIN, H), jnp.float32)] * NWIN
            + [pltpu.SemaphoreType.DMA] * (2 * NWIN + 1)),
    )(_sc_gather_body)
    return fn(ids_chunk, word_embeddings)


def _tc_ln_body(w_ref, p_ref, s_ref, prm_ref, o_ref, acc_ref=None):
    del acc_ref
    w = w_ref[...]
    p = p_ref[...]
    sg = s_ref[0].astype(jnp.float32)
    prm = prm_ref[...]
    g = prm[0:1, :]
    bb = prm[1:2, :]
    t0 = prm[2:3, :]
    td = prm[3:4, :] - t0
    e = w + p + t0 + sg * td
    mean = jnp.mean(e, axis=1, keepdims=True)
    cen = e - mean
    var = jnp.mean(cen * cen, axis=1, keepdims=True)
    o_ref[0] = cen * lax.rsqrt(var + EPS) * g + bb


def _tc_ln_body_acc(a_ref, w_ref, p_ref, s_ref, prm_ref, o_ref):
    _tc_ln_body(w_ref, p_ref, s_ref, prm_ref, o_ref, acc_ref=a_ref)


def _tc_ln(chunk, acc, wrows, segf, pos, params):
    grid = (BATCH,)
    data_specs = [
        pl.BlockSpec((TB, H), lambda i: (i, 0)),
        pl.BlockSpec((TB, H), lambda i, c=chunk: (c, 0)),
        pl.BlockSpec((1, TB, 1), lambda i, c=chunk: (i, c, 0)),
        pl.BlockSpec((4, H), lambda i: (0, 0)),
    ]
    out_spec = pl.BlockSpec((1, TB, H), lambda i, c=chunk: (i, c, 0))
    out_shape = jax.ShapeDtypeStruct((BATCH, L, H), jnp.float32)
    if chunk == 0:
        return pl.pallas_call(
            _tc_ln_body, grid=grid, in_specs=data_specs,
            out_specs=out_spec, out_shape=out_shape,
        )(wrows, pos, segf, params)
    return pl.pallas_call(
        _tc_ln_body_acc, grid=grid,
        in_specs=[pl.BlockSpec(memory_space=pl.ANY)] + data_specs,
        out_specs=out_spec, out_shape=out_shape,
        input_output_aliases={0: 0},
    )(acc, wrows, pos, segf, params)


@jax.jit
def _embed(input_ids, segment_ids, word_embeddings, position_embeddings,
           token_type_embeddings, ln_gamma, ln_beta):
    pos = position_embeddings[:L]
    params = jnp.concatenate(
        [ln_gamma.reshape(1, H), ln_beta.reshape(1, H),
         token_type_embeddings], axis=0)
    seg3 = segment_ids.reshape(BATCH, L, 1)
    ids = input_ids.reshape(BATCH, NCHUNK, LC)
    gathered = [
        _sc_gather(ids[:, c, :].reshape(TOK), word_embeddings)
        for c in range(NCHUNK)
    ]
    out = None
    for c in range(NCHUNK):
        out = _tc_ln(c, out, gathered[c], seg3, pos, params)
    return out


def kernel(input_ids, segment_ids, word_embeddings, position_embeddings,
           token_type_embeddings, ln_gamma, ln_beta):
    return _embed(input_ids.astype(jnp.int32), segment_ids.astype(jnp.int32),
                  word_embeddings, position_embeddings, token_type_embeddings,
                  ln_gamma, ln_beta)


# speedup vs baseline: 1.0612x; 1.0126x over previous
import functools

import jax
import jax.numpy as jnp
from jax import lax
from jax.experimental import pallas as pl
from jax.experimental.pallas import tpu as pltpu
from jax.experimental.pallas import tpu_sc as plsc

VOCAB = 30522
H = 768
BATCH = 4
L = 2048
EPS = 1e-12

NCHUNK = 4
LC = L // NCHUNK
TOK = BATCH * LC

NC = 2
NS = 16
NW = NC * NS
TPW = TOK // NW
SUB = TPW // 2
WPB = NW // BATCH

TB = LC


NWIN = 8
WIN = TPW // NWIN


def _sc_gather_body(ids_hbm, word_hbm, out_hbm, idx_v, *rest):
    bufs = rest[:NWIN]
    sem_i = rest[NWIN]
    gsems = rest[NWIN + 1:2 * NWIN + 1]
    osems = rest[2 * NWIN + 1:]
    c = lax.axis_index("c")
    s = lax.axis_index("s")
    wid = s * NC + c
    base = wid * TPW

    pltpu.async_copy(ids_hbm.at[pl.ds(base, TPW)], idx_v, sem_i).wait()
    gs = [
        pltpu.async_copy(
            word_hbm.at[idx_v.at[pl.ds(k * WIN, WIN)]], bufs[k], gsems[k])
        for k in range(NWIN)
    ]
    os = []
    for k in range(NWIN):
        gs[k].wait()
        os.append(pltpu.async_copy(
            bufs[k], out_hbm.at[pl.ds(base + k * WIN, WIN)], osems[k]))
    for o in os:
        o.wait()


def _sc_gather(ids_chunk, word_embeddings):
    fn = functools.partial(
        pl.kernel,
        mesh=plsc.VectorSubcoreMesh(core_axis_name="c", subcore_axis_name="s"),
        out_type=jax.ShapeDtypeStruct((TOK, H), jnp.float32),
        scratch_types=(
            [pltpu.VMEM((TPW,), jnp.int32)]
            + [pltpu.VMEM((WIN, H), jnp.float32)] * NWIN
            + [pltpu.SemaphoreType.DMA] * (2 * NWIN + 1)),
    )(_sc_gather_body)
    return fn(ids_chunk, word_embeddings)


def _tc_ln_body(w_ref, p_ref, s_ref, prm_ref, o_ref, acc_ref=None):
    del acc_ref
    w = w_ref[...]
    p = p_ref[...]
    sg = s_ref[0].astype(jnp.float32)
    prm = prm_ref[...]
    g = prm[0:1, :]
    bb = prm[1:2, :]
    t0 = prm[2:3, :]
    td = prm[3:4, :] - t0
    e = w + p + t0 + sg * td
    mean = jnp.mean(e, axis=1, keepdims=True)
    cen = e - mean
    var = jnp.mean(cen * cen, axis=1, keepdims=True)
    o_ref[0] = cen * lax.rsqrt(var + EPS) * g + bb


def _tc_ln_body_acc(a_ref, w_ref, p_ref, s_ref, prm_ref, o_ref):
    _tc_ln_body(w_ref, p_ref, s_ref, prm_ref, o_ref, acc_ref=a_ref)


def _tc_ln(chunk, acc, wrows, segf, pos, params):
    grid = (BATCH,)
    data_specs = [
        pl.BlockSpec((TB, H), lambda i: (i, 0)),
        pl.BlockSpec((TB, H), lambda i, c=chunk: (c, 0)),
        pl.BlockSpec((1, TB, 1), lambda i, c=chunk: (i, c, 0)),
        pl.BlockSpec((4, H), lambda i: (0, 0)),
    ]
    out_spec = pl.BlockSpec((1, TB, H), lambda i, c=chunk: (i, c, 0))
    out_shape = jax.ShapeDtypeStruct((BATCH, L, H), jnp.float32)
    if chunk == 0:
        return pl.pallas_call(
            _tc_ln_body, grid=grid, in_specs=data_specs,
            out_specs=out_spec, out_shape=out_shape,
        )(wrows, pos, segf, params)
    return pl.pallas_call(
        _tc_ln_body_acc, grid=grid,
        in_specs=[pl.BlockSpec(memory_space=pl.ANY)] + data_specs,
        out_specs=out_spec, out_shape=out_shape,
        input_output_aliases={0: 0},
    )(acc, wrows, pos, segf, params)


@jax.jit
def _embed(input_ids, segment_ids, word_embeddings, position_embeddings,
           token_type_embeddings, ln_gamma, ln_beta):
    pos = position_embeddings[:L]
    params = jnp.concatenate(
        [ln_gamma.reshape(1, H), ln_beta.reshape(1, H),
         token_type_embeddings], axis=0)
    seg3 = segment_ids.reshape(BATCH, L, 1)
    ids = input_ids.reshape(BATCH, NCHUNK, LC)
    gathered = [
        _sc_gather(ids[:, c, :].reshape(TOK), word_embeddings)
        for c in range(NCHUNK)
    ]
    out = None
    for c in range(NCHUNK):
        out = _tc_ln(c, out, gathered[c], seg3, pos, params)
    return out


def kernel(input_ids, segment_ids, word_embeddings, position_embeddings,
           token_type_embeddings, ln_gamma, ln_beta):
    return _embed(input_ids.astype(jnp.int32), segment_ids.astype(jnp.int32),
                  word_embeddings, position_embeddings, token_type_embeddings,
                  ln_gamma, ln_beta)
